# baseline (speedup 1.0000x reference)
import functools
import jax
import jax.numpy as jnp
from jax import lax
from jax.experimental import pallas as pl
from jax.experimental.pallas import tpu as pltpu
from jax.experimental.pallas import tpu_sc as plsc

NUM_ROWS = 16384
NUM_COLS = 1000
FAT_COLS = 896
TAIL_COLS = NUM_COLS - FAT_COLS
BLOCK_ROWS = 1024
NUM_WORKERS = 32
ROWS_PER_W = NUM_ROWS // NUM_WORKERS

_MESH = plsc.VectorSubcoreMesh(core_axis_name="c", subcore_axis_name="s")


@functools.partial(
    pl.kernel,
    out_type=jax.ShapeDtypeStruct((NUM_ROWS, NUM_COLS), jnp.float32),
    mesh=_MESH,
    scratch_types=[
        pltpu.VMEM((ROWS_PER_W,), jnp.int32),
        pltpu.VMEM((ROWS_PER_W, TAIL_COLS), jnp.float32),
    ],
    compiler_params=pltpu.CompilerParams(needs_layout_passes=False),
)
def _sc_tail(x_hbm, out_hbm, xv, blk):
    wid = lax.axis_index("s") * 2 + lax.axis_index("c")
    base = wid * ROWS_PER_W
    pltpu.sync_copy(x_hbm.at[pl.ds(base, ROWS_PER_W)], xv)

    zeros16 = jnp.zeros((16,), jnp.float32)
    ones16 = jnp.ones((16,), jnp.float32)
    iota16 = lax.iota(jnp.int32, 16)

    def zero_row(r, carry):
        for c0 in range(0, 96, 16):
            blk[r, pl.ds(c0, 16)] = zeros16
        plsc.store_scatter(
            blk,
            [jnp.full((16,), r, jnp.int32), iota16 + 96],
            zeros16,
            mask=iota16 < (TAIL_COLS - 96),
        )
        return carry

    lax.fori_loop(0, ROWS_PER_W, zero_row, 0)

    def poke(i, carry):
        xvv = xv[pl.ds(i * 16, 16)]
        ridx = i * 16 + iota16
        cidx = xvv - FAT_COLS
        plsc.store_scatter(blk, [ridx, cidx], ones16, mask=xvv >= FAT_COLS)
        return carry

    lax.fori_loop(0, ROWS_PER_W // 16, poke, 0)

    pltpu.sync_copy(
        blk,
        out_hbm.at[pl.ds(base, ROWS_PER_W), pl.ds(FAT_COLS, TAIL_COLS)],
    )


def _fat_body(x_ref, prev_ref, o_ref):
    i = pl.program_id(0)
    xs = x_ref[0, pl.ds(i * BLOCK_ROWS, BLOCK_ROWS)]
    cols = lax.broadcasted_iota(jnp.int32, (BLOCK_ROWS, FAT_COLS), 1)
    o_ref[...] = (cols == xs[:, None]).astype(jnp.float32)


def kernel(x):
    xi = x.astype(jnp.int32)
    out0 = _sc_tail(xi)
    x2 = xi.reshape(1, NUM_ROWS)
    out = pl.pallas_call(
        _fat_body,
        grid=(NUM_ROWS // BLOCK_ROWS,),
        in_specs=[
            pl.BlockSpec((1, NUM_ROWS), lambda i: (0, 0)),
            pl.BlockSpec(memory_space=pl.ANY),
        ],
        out_specs=pl.BlockSpec((BLOCK_ROWS, FAT_COLS), lambda i: (i, 0)),
        out_shape=jax.ShapeDtypeStruct((NUM_ROWS, NUM_COLS), jnp.float32),
        input_output_aliases={1: 0},
    )(x2, out0)
    return out


# speedup vs baseline: 1.0026x; 1.0026x over previous
import functools
import jax
import jax.numpy as jnp
from jax import lax
from jax.experimental import pallas as pl
from jax.experimental.pallas import tpu as pltpu
from jax.experimental.pallas import tpu_sc as plsc

NUM_ROWS = 16384
NUM_COLS = 1000
FAT_COLS = 896
TAIL_COLS = NUM_COLS - FAT_COLS
BLOCK_ROWS = 1024
NUM_WORKERS = 32
ROWS_PER_W = NUM_ROWS // NUM_WORKERS

_MESH = plsc.VectorSubcoreMesh(core_axis_name="c", subcore_axis_name="s")


@functools.partial(
    pl.kernel,
    out_type=jax.ShapeDtypeStruct((NUM_ROWS, NUM_COLS), jnp.float32),
    mesh=_MESH,
    scratch_types=[
        pltpu.VMEM((ROWS_PER_W,), jnp.int32),
        pltpu.VMEM((ROWS_PER_W, TAIL_COLS), jnp.float32),
    ],
    compiler_params=pltpu.CompilerParams(needs_layout_passes=False),
)
def _sc_tail(x_hbm, out_hbm, xv, blk):
    wid = lax.axis_index("s") * 2 + lax.axis_index("c")
    base = wid * ROWS_PER_W
    pltpu.sync_copy(x_hbm.at[pl.ds(base, ROWS_PER_W)], xv)

    zeros16 = jnp.zeros((16,), jnp.float32)
    ones16 = jnp.ones((16,), jnp.float32)
    iota16 = lax.iota(jnp.int32, 16)

    def zero_row(r, carry):
        for c0 in range(0, 96, 16):
            blk[r, pl.ds(c0, 16)] = zeros16
        plsc.store_scatter(
            blk,
            [jnp.full((16,), r, jnp.int32), iota16 + 96],
            zeros16,
            mask=iota16 < (TAIL_COLS - 96),
        )
        return carry

    lax.fori_loop(0, ROWS_PER_W, zero_row, 0)

    def poke(i, carry):
        xvv = xv[pl.ds(i * 16, 16)]
        ridx = i * 16 + iota16
        cidx = xvv - FAT_COLS
        plsc.store_scatter(blk, [ridx, cidx], ones16, mask=xvv >= FAT_COLS)
        return carry

    lax.fori_loop(0, ROWS_PER_W // 16, poke, 0)

    pltpu.sync_copy(
        blk,
        out_hbm.at[pl.ds(base, ROWS_PER_W), pl.ds(FAT_COLS, TAIL_COLS)],
    )


FB_ROWS = 512
FB_SLOTS = 8
FB_CHUNKS = NUM_ROWS // FB_ROWS
FB_ROUNDS = FB_CHUNKS // FB_SLOTS


def _fat_copy(o_ref, buf_ref, sem_ref, k, ci):
    return pltpu.make_async_copy(
        buf_ref.at[k],
        o_ref.at[pl.ds(ci * FB_ROWS, FB_ROWS), pl.ds(0, FAT_COLS)],
        sem_ref.at[k],
    )


def _fat_body(x_ref, prev_ref, o_ref, buf_ref, sem_ref):
    def one_round(r, carry):
        for k in range(FB_SLOTS):
            ci = r * FB_SLOTS + k

            @pl.when(r > 0)
            def _wait_prev():
                _fat_copy(o_ref, buf_ref, sem_ref, k, ci).wait()

            xs = x_ref[0, pl.ds(ci * FB_ROWS, FB_ROWS)]
            cols = lax.broadcasted_iota(jnp.int32, (FB_ROWS, FAT_COLS), 1)
            buf_ref[k] = (cols == xs[:, None]).astype(jnp.float32)
            _fat_copy(o_ref, buf_ref, sem_ref, k, ci).start()
        return carry

    lax.fori_loop(0, FB_ROUNDS, one_round, 0)
    for k in range(FB_SLOTS):
        ci = (FB_ROUNDS - 1) * FB_SLOTS + k
        _fat_copy(o_ref, buf_ref, sem_ref, k, ci).wait()


def kernel(x):
    xi = x.astype(jnp.int32)
    out0 = _sc_tail(xi)
    x2 = xi.reshape(1, NUM_ROWS)
    out = pl.pallas_call(
        _fat_body,
        in_specs=[
            pl.BlockSpec(memory_space=pltpu.VMEM),
            pl.BlockSpec(memory_space=pl.ANY),
        ],
        out_specs=pl.BlockSpec(memory_space=pl.ANY),
        out_shape=jax.ShapeDtypeStruct((NUM_ROWS, NUM_COLS), jnp.float32),
        scratch_shapes=[
            pltpu.VMEM((FB_SLOTS, FB_ROWS, FAT_COLS), jnp.float32),
            pltpu.SemaphoreType.DMA((FB_SLOTS,)),
        ],
        input_output_aliases={1: 0},
    )(x2, out0)
    return out
